# baseline (speedup 1.0000x reference)
import functools

import jax
import jax.numpy as jnp
from jax import lax
from jax.experimental import pallas as pl
from jax.experimental.pallas import tpu as pltpu
from jax.experimental.pallas import tpu_sc as plsc

H = 224
C = 96
R = 16
NSTRIP = H // R
NCORES = 2
NSUB = 16
NW = NCORES * NSUB
CPW = C // NW
TPW = CPW * NSTRIP
NCHUNK = H // 16
PLANE = H * H
STRIPW = R * H
PAD = 240
BUFW = PAD + STRIPW + PAD
STMAX = (H - R) * H


def _build_sc_call():
    mesh = plsc.VectorSubcoreMesh(core_axis_name="c", subcore_axis_name="s")

    @functools.partial(
        pl.kernel,
        out_type=jax.ShapeDtypeStruct((C, H, H), jnp.float32),
        mesh=mesh,
        compiler_params=pltpu.CompilerParams(use_tc_tiling_on_sc=False),
        scratch_types=[
            pltpu.VMEM((2, 9, BUFW), jnp.float32),
            pltpu.VMEM((2, R, H), jnp.float32),
            pltpu.SemaphoreType.DMA,
            pltpu.SemaphoreType.DMA,
            pltpu.SemaphoreType.DMA,
            pltpu.SemaphoreType.DMA,
        ],
    )
    def col2im_sc(x_hbm, out_hbm, ibuf, obuf, isem0, isem1, osem0, osem1):
        wid = lax.axis_index("s") * NCORES + lax.axis_index("c")
        base_t = wid * TPW
        isem = (isem0, isem1)
        osem = (osem0, osem1)
        zeros16 = jnp.zeros((16,), jnp.float32)
        lane_f = lax.iota(jnp.int32, 16).astype(jnp.float32)
        mask_lo = jnp.minimum(lane_f, 1.0)
        mask_hi = jnp.minimum(15.0 - lane_f, 1.0)

        def zpad(i, carry):
            for b in range(2):
                for p in range(9):
                    ibuf[b, p, pl.ds(i * 16, 16)] = zeros16
                    ibuf[b, p, pl.ds(PAD + STRIPW + i * 16, 16)] = zeros16
            return carry

        lax.fori_loop(0, PAD // 16, zpad, 0)

        def split(t):
            c = t // NSTRIP
            s = t - c * NSTRIP
            return c, s

        def starts(t):
            _, s = split(t)
            h0 = s * R
            st = []
            for kh in range(3):
                raw = (h0 + 1 - kh) * H
                st.append(pl.multiple_of(jnp.clip(raw, 0, STMAX), H))
            return st

        def in_copies(t, b):
            c, _ = split(t)
            st = starts(t)
            cps = []
            for p in range(9):
                kh = p // 3
                cps.append(pltpu.make_async_copy(
                    x_hbm.at[0, c * 9 + p, pl.ds(st[kh], STRIPW)],
                    ibuf.at[b, p, pl.ds(PAD, STRIPW)],
                    isem[b]))
            return cps

        def out_copy(t, b):
            c, s = split(t)
            return pltpu.make_async_copy(
                obuf.at[b], out_hbm.at[c, pl.ds(s * R, R), :], osem[b])

        def issue_in(t, b):
            for cp in in_copies(t, b):
                cp.start()

        def wait_in(t, b):
            for cp in in_copies(t, b):
                cp.wait()

        def compute(t, b):
            _, s = split(t)
            h0 = s * R
            st = starts(t)
            rb = [PAD + (h0 + 1 - kh) * H - st[kh] for kh in range(3)]

            def rowf(r, cc):
                rowbase = r * H
                b0 = rb[0] + rowbase
                b1 = rb[1] + rowbase
                b2 = rb[2] + rowbase
                for ch in range(NCHUNK):
                    o0 = ch * 16 + 1
                    o1 = ch * 16
                    o2 = ch * 16 - 1
                    t0 = ibuf[b, 0, pl.ds(b0 + o0, 16)]
                    t3 = ibuf[b, 3, pl.ds(b1 + o0, 16)]
                    t6 = ibuf[b, 6, pl.ds(b2 + o0, 16)]
                    acc0 = t0 + t3 + t6
                    if ch == NCHUNK - 1:
                        acc0 = acc0 * mask_hi
                    t1 = ibuf[b, 1, pl.ds(b0 + o1, 16)]
                    t4 = ibuf[b, 4, pl.ds(b1 + o1, 16)]
                    t7 = ibuf[b, 7, pl.ds(b2 + o1, 16)]
                    acc1 = t1 + t4 + t7
                    t2 = ibuf[b, 2, pl.ds(b0 + o2, 16)]
                    t5 = ibuf[b, 5, pl.ds(b1 + o2, 16)]
                    t8 = ibuf[b, 8, pl.ds(b2 + o2, 16)]
                    acc2 = t2 + t5 + t8
                    if ch == 0:
                        acc2 = acc2 * mask_lo
                    obuf[b, r, pl.ds(ch * 16, 16)] = acc0 + acc1 + acc2
                return cc

            lax.fori_loop(0, R, rowf, 0)

        issue_in(base_t, 0)

        def pair(g, carry):
            t0 = base_t + 2 * g
            issue_in(t0 + 1, 1)
            wait_in(t0, 0)

            @pl.when(g > 0)
            def _():
                out_copy(t0, 0).wait()

            compute(t0, 0)
            out_copy(t0, 0).start()

            @pl.when(g < TPW // 2 - 1)
            def _():
                issue_in(t0 + 2, 0)

            wait_in(t0 + 1, 1)

            @pl.when(g > 0)
            def _():
                out_copy(t0 + 1, 1).wait()

            compute(t0 + 1, 1)
            out_copy(t0 + 1, 1).start()
            return carry

        lax.fori_loop(0, TPW // 2, pair, 0)
        out_copy(base_t, 0).wait()
        out_copy(base_t, 1).wait()

    return col2im_sc


_COL2IM_SC = _build_sc_call()


def kernel(x, output_size, kernel_size, dilation, padding, stride):
    out = _COL2IM_SC(x)
    return out.reshape(1, C, H, H)


# speedup vs baseline: 1.4668x; 1.4668x over previous
import functools

import jax
import jax.numpy as jnp
from jax import lax
from jax.experimental import pallas as pl
from jax.experimental.pallas import tpu as pltpu
from jax.experimental.pallas import tpu_sc as plsc

H = 224
C = 96
R = 16
NSTRIP = H // R
NCORES = 2
NSUB = 16
NW = NCORES * NSUB
CPW = C // NW
TPW = CPW * NSTRIP
NCHUNK = H // 16
PLANE = H * H
STRIPW = R * H
NTI = (C * 9) // 8
NTJ = PLANE // 128
STAGE = 32
BUFT = 36
TJMAX = NTJ - STAGE


def _build_sc_call():
    mesh = plsc.VectorSubcoreMesh(core_axis_name="c", subcore_axis_name="s")

    @functools.partial(
        pl.kernel,
        out_type=jax.ShapeDtypeStruct((C, H, H), jnp.float32),
        mesh=mesh,
        compiler_params=pltpu.CompilerParams(
            use_tc_tiling_on_sc=False, needs_layout_passes=False),
        scratch_types=[
            pltpu.VMEM((2, 9, BUFT, 128), jnp.float32),
            pltpu.VMEM((2, R, H), jnp.float32),
            pltpu.SemaphoreType.DMA,
            pltpu.SemaphoreType.DMA,
            pltpu.SemaphoreType.DMA,
            pltpu.SemaphoreType.DMA,
        ],
    )
    def col2im_sc(x_hbm, out_hbm, ibuf, obuf, isem0, isem1, osem0, osem1):
        wid = lax.axis_index("s") * NCORES + lax.axis_index("c")
        base_t = wid * TPW
        isem = (isem0, isem1)
        osem = (osem0, osem1)
        zeros16 = jnp.zeros((16,), jnp.float32)
        lane = lax.iota(jnp.int32, 16)
        lane_f = lane.astype(jnp.float32)
        mask_lo = jnp.minimum(lane_f, 1.0)
        mask_hi = jnp.minimum(15.0 - lane_f, 1.0)

        def zpad(i, carry):
            for bb in range(2):
                for p in range(9):
                    for tt in (0, 1, BUFT - 2, BUFT - 1):
                        ibuf[bb, p, tt, pl.ds(i * 16, 16)] = zeros16
            return carry

        lax.fori_loop(0, 128 // 16, zpad, 0)

        def split(t):
            c = t // NSTRIP
            s = t - c * NSTRIP
            return c, s

        def in_copies(t, bb):
            c, s = split(t)
            b = s * STRIPW
            tja = jnp.clip(b - 256, 0, TJMAX * 128) // 128
            cps = []
            for p in range(9):
                q = c * 9 + p
                ti = q // 8
                ii = q - ti * 8
                cps.append(pltpu.make_async_copy(
                    x_hbm.at[ti, pl.ds(tja, STAGE), ii, :],
                    ibuf.at[bb, p, pl.ds(2, STAGE), :],
                    isem[bb]))
            return cps

        def out_copy(t, bb):
            c, s = split(t)
            return pltpu.make_async_copy(
                obuf.at[bb], out_hbm.at[c, pl.ds(s * R, R), :], osem[bb])

        def issue_in(t, bb):
            for cp in in_copies(t, bb):
                cp.start()

        def wait_in(t, bb):
            for cp in in_copies(t, bb):
                cp.wait()

        def compute(t, bb):
            _, s = split(t)
            b = s * STRIPW
            tja = jnp.clip(b - 256, 0, TJMAX * 128) // 128
            A = 256 + b - tja * 128
            ashift = A // 128

            def gbody(g, cc):
                sg = ashift + 7 * g
                rowvec = [A + g * 896 + j * 224 + lane for j in range(4)]
                for j in range(4):
                    r = 4 * g + j
                    for ch in range(NCHUNK):
                        zo = j * 224 + ch * 16
                        accs = []
                        for kw in (0, 1, 2):
                            acc = None
                            for kh in (0, 1, 2):
                                p = kh * 3 + kw
                                d = (1 - kh) * 224 + (1 - kw)
                                if kw == 1:
                                    off = zo + d
                                    v = ibuf[bb, p, sg + off // 128,
                                             pl.ds(off % 128, 16)]
                                else:
                                    idx = rowvec[j] + (ch * 16 + d)
                                    t_idx = lax.shift_right_logical(idx, 7)
                                    c_idx = lax.bitwise_and(idx, 127)
                                    v = plsc.load_gather(
                                        ibuf.at[bb, p], [t_idx, c_idx])
                                acc = v if acc is None else acc + v
                            accs.append(acc)
                        a0, a1, a2 = accs
                        if ch == NCHUNK - 1:
                            a0 = a0 * mask_hi
                        if ch == 0:
                            a2 = a2 * mask_lo
                        obuf[bb, r, pl.ds(ch * 16, 16)] = a0 + a1 + a2
                return cc

            lax.fori_loop(0, 4, gbody, 0)

        issue_in(base_t, 0)

        def pair(g, carry):
            t0 = base_t + 2 * g
            issue_in(t0 + 1, 1)
            wait_in(t0, 0)

            @pl.when(g > 0)
            def _():
                out_copy(t0, 0).wait()

            compute(t0, 0)
            out_copy(t0, 0).start()

            @pl.when(g < TPW // 2 - 1)
            def _():
                issue_in(t0 + 2, 0)

            wait_in(t0 + 1, 1)

            @pl.when(g > 0)
            def _():
                out_copy(t0 + 1, 1).wait()

            compute(t0 + 1, 1)
            out_copy(t0 + 1, 1).start()
            return carry

        lax.fori_loop(0, TPW // 2, pair, 0)
        out_copy(base_t, 0).wait()
        out_copy(base_t, 1).wait()

    return col2im_sc


_COL2IM_SC = _build_sc_call()


def kernel(x, output_size, kernel_size, dilation, padding, stride):
    x4t = lax.transpose(x.reshape(NTI, 8, NTJ, 128), (0, 2, 1, 3))
    out = _COL2IM_SC(x4t)
    return out.reshape(1, C, H, H)
